# baseline (speedup 1.0000x reference)
import functools

import jax
import jax.numpy as jnp
from jax import lax
from jax.experimental import pallas as pl
from jax.experimental.pallas import tpu as pltpu
from jax.experimental.pallas import tpu_sc as plsc

B, S, D = 4, 4096, 1024
L = 16
PPR = D // L

_info = plsc.get_sparse_core_info()
NC, NS = _info.num_cores, _info.num_subcores
NW = NC * NS
S_PER_W = S // NW
CHUNK = 8
N_CHUNKS = S_PER_W // CHUNK
DEPTH = 3

_mesh = plsc.VectorSubcoreMesh(core_axis_name="c", subcore_axis_name="s")


@functools.partial(
    pl.kernel,
    mesh=_mesh,
    out_type=jax.ShapeDtypeStruct((B * S, D), jnp.float32),
    scratch_types=(
        [pltpu.VMEM((CHUNK, D), jnp.float32) for _ in range(DEPTH)]
        + [pltpu.VMEM((CHUNK, D), jnp.float32) for _ in range(DEPTH * B)]
        + [pltpu.SemaphoreType.DMA for _ in range(DEPTH * (1 + 2 * B))]
    ),
)
def _sc_add(x_hbm, pos_hbm, out_hbm, *scr):
    pos_bufs = list(scr[0:DEPTH])
    x_bufs = [list(scr[DEPTH + p * B:DEPTH + (p + 1) * B])
              for p in range(DEPTH)]
    sems = scr[DEPTH + DEPTH * B:]
    pos_sems = list(sems[0:DEPTH])
    in_sems = [list(sems[DEPTH + p * B:DEPTH + (p + 1) * B])
               for p in range(DEPTH)]
    off = DEPTH + DEPTH * B
    out_sems = [list(sems[off + p * B:off + (p + 1) * B])
                for p in range(DEPTH)]

    wid = lax.axis_index("s") * NC + lax.axis_index("c")
    s_base = wid * S_PER_W

    def start_pos(c):
        p = c % DEPTH
        return pltpu.async_copy(
            pos_hbm.at[pl.ds(s_base + c * CHUNK, CHUNK)], pos_bufs[p],
            pos_sems[p])

    def start_in(c, b):
        p = c % DEPTH
        row = b * S + s_base + c * CHUNK
        return pltpu.async_copy(
            x_hbm.at[pl.ds(row, CHUNK)], x_bufs[p][b], in_sems[p][b])

    def start_out(c, b):
        p = c % DEPTH
        row = b * S + s_base + c * CHUNK
        return pltpu.async_copy(
            x_bufs[p][b], out_hbm.at[pl.ds(row, CHUNK)], out_sems[p][b])

    pos_h = [None] * DEPTH
    in_h = [[None] * B for _ in range(DEPTH)]
    out_h = [[None] * B for _ in range(DEPTH)]
    for c in range(DEPTH - 1):
        pos_h[c] = start_pos(c)
        for b in range(B):
            in_h[c][b] = start_in(c, b)

    for c in range(N_CHUNKS):
        par = c % DEPTH
        nc = c + DEPTH - 1
        if nc < N_CHUNKS:
            npar = nc % DEPTH
            for b in range(B):
                if out_h[npar][b] is not None:
                    out_h[npar][b].wait()
                in_h[npar][b] = start_in(nc, b)
            pos_h[npar] = start_pos(nc)
        pos_h[par].wait()
        for b in range(B):
            in_h[par][b].wait()

        xb, pos_v = x_bufs[par], pos_bufs[par]

        @plsc.parallel_loop(0, CHUNK * PPR, step=1, unroll=4)
        def _add(i):
            r = i // PPR
            col = (i % PPR) * L
            p = pos_v[r, pl.ds(col, L)]
            for b in range(B):
                plsc.addupdate(xb[b].at[r, pl.ds(col, L)], p)

        for b in range(B):
            out_h[par][b] = start_out(c, b)

    for hs in out_h:
        for h in hs:
            if h is not None:
                h.wait()


def kernel(x, pos_table):
    out = _sc_add(x.reshape(B * S, D), pos_table)
    return out.reshape(x.shape)
